# baseline (speedup 1.0000x reference)
import functools

import jax
import jax.numpy as jnp
from jax import lax
from jax.experimental import pallas as pl
from jax.experimental.pallas import tpu as pltpu
from jax.experimental.pallas import tpu_sc as plsc

N = 10000
D = 128
E = 320000
C = 1.0
MIN_NORM = 1e-15
PROJ_EPS = 4e-3
MAXNORM = 1.0 - PROJ_EPS

NC = 2
NS = 16
NW = NC * NS
K = 64
NCHUNK = 160
EPT = NCHUNK * K
E_PAD = NW * EPT
N_PAD = 10240
ROWS_PER_TILE = N_PAD // NS


def _artanh(v):
    v = jnp.clip(v, -1.0 + 1e-7, 1.0 - 1e-7)
    return 0.5 * jnp.log((1.0 + v) / (1.0 - v))


def _norm(x):
    return jnp.maximum(jnp.sqrt(jnp.sum(x * x, axis=-1, keepdims=True)), MIN_NORM)


def _proj(x):
    norm = _norm(x)
    return jnp.where(norm > MAXNORM, x / norm * MAXNORM, x)


def _expmap0(u):
    un = _norm(u)
    return jnp.tanh(un) * u / un


def _logmap0(p):
    pn = _norm(p)
    return _artanh(pn) * p / pn


def _stage_a_body(x_ref, w_ref, b_ref, o_ref):
    x = x_ref[...]
    W = w_ref[...]
    b = b_ref[...]
    xn = _norm(x)
    mx = lax.dot_general(x, W, (((1,), (1,)), ((), ())),
                         preferred_element_type=jnp.float32)
    mxn = _norm(mx)
    res = jnp.tanh(mxn / xn * _artanh(xn)) * mx / mxn
    res = jnp.where(jnp.all(mx == 0.0, axis=-1, keepdims=True),
                    jnp.zeros_like(res), res)
    res = _proj(res)
    hyp_bias = _proj(_expmap0(b))
    x2 = jnp.sum(res * res, axis=-1, keepdims=True)
    y2 = jnp.sum(hyp_bias * hyp_bias, axis=-1, keepdims=True)
    xy = jnp.sum(res * hyp_bias, axis=-1, keepdims=True)
    num = (1.0 + 2.0 * xy + y2) * res + (1.0 - x2) * hyp_bias
    den = 1.0 + 2.0 * xy + x2 * y2
    h = _proj(num / jnp.maximum(den, MIN_NORM))
    o_ref[...] = _logmap0(h)


def _stage_c_body(a_ref, o_ref):
    s = a_ref[0] + a_ref[1]
    h = _proj(_expmap0(s))
    xt = jnp.maximum(_logmap0(h), 0.0)
    o_ref[...] = _proj(_expmap0(xt))


_BLK = 2000


def _stage_a(x, W, b2d):
    return pl.pallas_call(
        _stage_a_body,
        grid=(N // _BLK,),
        in_specs=[
            pl.BlockSpec((_BLK, D), lambda i: (i, 0)),
            pl.BlockSpec((D, D), lambda i: (0, 0)),
            pl.BlockSpec((1, D), lambda i: (0, 0)),
        ],
        out_specs=pl.BlockSpec((_BLK, D), lambda i: (i, 0)),
        out_shape=jax.ShapeDtypeStruct((N, D), jnp.float32),
    )(x, W, b2d)


def _stage_c(parts):
    return pl.pallas_call(
        _stage_c_body,
        grid=(N // _BLK,),
        in_specs=[pl.BlockSpec((2, _BLK, D), lambda i: (0, i, 0))],
        out_specs=pl.BlockSpec((_BLK, D), lambda i: (i, 0)),
        out_shape=jax.ShapeDtypeStruct((N, D), jnp.float32),
    )(parts)


@functools.cache
def _build_sc_agg():
    mesh = plsc.VectorSubcoreMesh(
        core_axis_name="c", subcore_axis_name="s",
        num_cores=NC, num_subcores=NS)
    return functools.partial(
        pl.kernel,
        out_type=jax.ShapeDtypeStruct((NC, N_PAD, D), jnp.float32),
        mesh=mesh,
        compiler_params=pltpu.CompilerParams(needs_layout_passes=False),
        scratch_types=[
            pltpu.VMEM((NCHUNK // 2, 128), jnp.int32),
            pltpu.VMEM((4, K), jnp.int32),
            pltpu.VMEM((4, K), jnp.int32),
            pltpu.VMEM((4, 1, K * 16), jnp.float32),
            pltpu.VMEM((K, D), jnp.float32),
            pltpu.VMEM((K, D), jnp.float32),
            pltpu.VMEM((K, D), jnp.float32),
            pltpu.VMEM((K, D), jnp.float32),
            pltpu.VMEM_SHARED((N_PAD, D), jnp.float32),
            pltpu.SemaphoreType.DMA,
            pltpu.SemaphoreType.DMA,
            pltpu.SemaphoreType.DMA,
            pltpu.SemaphoreType.DMA,
            pltpu.SemaphoreType.DMA,
            pltpu.SemaphoreType.DMA,
            pltpu.SemaphoreType.DMA,
            pltpu.SemaphoreType.DMA,
            pltpu.SemaphoreType.DMA,
            pltpu.SemaphoreType.DMA,
            pltpu.SemaphoreType.DMA,
            pltpu.SemaphoreType.DMA,
        ],
    )(_sc_agg_body)


def _sc_agg_body(xt_hbm, packed_hbm, w_hbm, zeros_hbm, out_hbm,
                 packed_v, colbuf, rowbuf, wring, b0, b1, b2, b3, acc,
                 g0, g1, g2, g3, s0, s1, s2, s3, w0, w1, w2, w3):
    bufs = (b0, b1, b2, b3)
    gsem = (g0, g1, g2, g3)
    ssem = (s0, s1, s2, s3)
    wsem = (w0, w1, w2, w3)
    c = lax.axis_index("c")
    s = lax.axis_index("s")
    wid = s * NC + c
    pltpu.sync_copy(packed_hbm.at[pl.ds(wid * (NCHUNK // 2), NCHUNK // 2)],
                    packed_v)
    pltpu.sync_copy(zeros_hbm, acc.at[pl.ds(s * ROWS_PER_TILE, ROWS_PER_TILE)])
    plsc.subcore_barrier()

    def wait_scatter(r):
        pltpu.make_async_copy(bufs[r], acc.at[rowbuf.at[r]], ssem[r]).wait()

    def prefetch(p, r):
        base = (p % 2) * K

        @pl.loop(0, K // 16)
        def _unpack(g):
            sl = pl.ds(g * 16, 16)
            v = packed_v[p // 2, pl.ds(base + g * 16, 16)]
            colbuf[r, sl] = lax.bitwise_and(v, 0xFFFF)
            rowbuf[r, sl] = lax.shift_right_logical(v, 16)

        pltpu.async_copy(w_hbm.at[wid * NCHUNK + p], wring.at[r], wsem[r])
        pltpu.async_copy(xt_hbm.at[colbuf.at[r]], bufs[r], gsem[r])

    prefetch(0, 0)
    prefetch(1, 1)

    @pl.loop(0, NCHUNK, step=4)
    def _group(i0):
        for b in range(4):
            i = i0 + b
            r2 = (b + 2) % 4

            @pl.when(i + 2 < NCHUNK)
            def _prefetch():
                @pl.when(i >= 2)
                def _pf_drain():
                    wait_scatter(r2)
                prefetch(i + 2, r2)

            pltpu.make_async_copy(
                xt_hbm.at[colbuf.at[b]], bufs[b], gsem[b]).wait()
            pltpu.make_async_copy(
                w_hbm.at[wid * NCHUNK], wring.at[b], wsem[b]).wait()

            @pl.loop(0, K, unroll=4)
            def _edge(j):
                wv = wring[b, 0, pl.ds(j * 16, 16)]
                for d in range(D // 16):
                    sl = pl.ds(d * 16, 16)
                    bufs[b][j, sl] = bufs[b][j, sl] * wv

            pltpu.async_copy(bufs[b], acc.at[rowbuf.at[b]], ssem[b], add=True)

    for b in range(4):
        wait_scatter(b)
    plsc.subcore_barrier()
    pltpu.sync_copy(acc.at[pl.ds(s * ROWS_PER_TILE, ROWS_PER_TILE)],
                    out_hbm.at[c, pl.ds(s * ROWS_PER_TILE, ROWS_PER_TILE)])


def kernel(x, edge_index, edge_weight, W, b):
    xt = _stage_a(x, W, b.reshape(1, D))
    pad = E_PAD - E
    row = jnp.concatenate([edge_index[0], jnp.zeros((pad,), jnp.int32)])
    col = jnp.concatenate([edge_index[1], jnp.zeros((pad,), jnp.int32)])
    w = jnp.concatenate([edge_weight, jnp.zeros((pad,), jnp.float32)])
    packed = (col | (row << 16)).reshape(NW * NCHUNK // 2, 128)
    w3 = jnp.broadcast_to(w[:, None], (E_PAD, 16)).reshape(NW * NCHUNK, 1, K * 16)
    zeros = jnp.zeros((ROWS_PER_TILE, D), jnp.float32)
    parts = _build_sc_agg()(xt, packed, w3, zeros)
    return _stage_c(parts)


# speedup vs baseline: 2.0522x; 2.0522x over previous
import functools

import jax
import jax.numpy as jnp
from jax import lax
from jax.experimental import pallas as pl
from jax.experimental.pallas import tpu as pltpu
from jax.experimental.pallas import tpu_sc as plsc

N = 10000
D = 128
E = 320000
MIN_NORM = 1e-15
PROJ_EPS = 4e-3
MAXNORM = 1.0 - PROJ_EPS

NC = 2
NS = 16
K = 64
NCH0 = 264
NCH1 = 52
TOTCH = NS * (NCH0 + NCH1)
E_PAD = TOTCH * K
N_PAD = 10240
ROWS_PER_TILE = N_PAD // NS


def _artanh(v):
    v = jnp.clip(v, -1.0 + 1e-7, 1.0 - 1e-7)
    return 0.5 * jnp.log((1.0 + v) / (1.0 - v))


def _norm(x):
    return jnp.maximum(jnp.sqrt(jnp.sum(x * x, axis=-1, keepdims=True)), MIN_NORM)


def _proj(x):
    norm = _norm(x)
    return jnp.where(norm > MAXNORM, x / norm * MAXNORM, x)


def _expmap0(u):
    un = _norm(u)
    return jnp.tanh(un) * u / un


def _logmap0(p):
    pn = _norm(p)
    return _artanh(pn) * p / pn


def _stage_a_body(x_ref, w_ref, b_ref, o_ref):
    x = x_ref[...]
    W = w_ref[...]
    b = b_ref[...]
    xn = _norm(x)
    mx = lax.dot_general(x, W, (((1,), (1,)), ((), ())),
                         preferred_element_type=jnp.float32)
    mxn = _norm(mx)
    res = jnp.tanh(mxn / xn * _artanh(xn)) * mx / mxn
    res = jnp.where(jnp.all(mx == 0.0, axis=-1, keepdims=True),
                    jnp.zeros_like(res), res)
    res = _proj(res)
    hyp_bias = _proj(_expmap0(b))
    x2 = jnp.sum(res * res, axis=-1, keepdims=True)
    y2 = jnp.sum(hyp_bias * hyp_bias, axis=-1, keepdims=True)
    xy = jnp.sum(res * hyp_bias, axis=-1, keepdims=True)
    num = (1.0 + 2.0 * xy + y2) * res + (1.0 - x2) * hyp_bias
    den = 1.0 + 2.0 * xy + x2 * y2
    h = _proj(num / jnp.maximum(den, MIN_NORM))
    o_ref[...] = _logmap0(h)


def _stage_c_body(a_ref, o_ref):
    s = a_ref[0] + a_ref[1]
    h = _proj(_expmap0(s))
    xt = jnp.maximum(_logmap0(h), 0.0)
    o_ref[...] = _proj(_expmap0(xt))


_BLK = 2000


def _stage_a(x, W, b2d):
    return pl.pallas_call(
        _stage_a_body,
        grid=(N // _BLK,),
        in_specs=[
            pl.BlockSpec((_BLK, D), lambda i: (i, 0)),
            pl.BlockSpec((D, D), lambda i: (0, 0)),
            pl.BlockSpec((1, D), lambda i: (0, 0)),
        ],
        out_specs=pl.BlockSpec((_BLK, D), lambda i: (i, 0)),
        out_shape=jax.ShapeDtypeStruct((N, D), jnp.float32),
    )(x, W, b2d)


def _stage_c(parts):
    return pl.pallas_call(
        _stage_c_body,
        grid=(N // _BLK,),
        in_specs=[pl.BlockSpec((2, _BLK, D), lambda i: (0, i, 0))],
        out_specs=pl.BlockSpec((_BLK, D), lambda i: (i, 0)),
        out_shape=jax.ShapeDtypeStruct((N, D), jnp.float32),
    )(parts)


@functools.cache
def _build_sc_agg():
    mesh = plsc.VectorSubcoreMesh(
        core_axis_name="c", subcore_axis_name="s",
        num_cores=NC, num_subcores=NS)
    return functools.partial(
        pl.kernel,
        out_type=jax.ShapeDtypeStruct((NC, N_PAD, D), jnp.float32),
        mesh=mesh,
        compiler_params=pltpu.CompilerParams(needs_layout_passes=False),
        scratch_types=[
            pltpu.VMEM((4, 1, K), jnp.int32),
            pltpu.VMEM((4, K), jnp.int32),
            pltpu.VMEM((4, K), jnp.int32),
            pltpu.VMEM((4, 1, K), jnp.float32),
            pltpu.VMEM((K, D), jnp.float32),
            pltpu.VMEM((K, D), jnp.float32),
            pltpu.VMEM((K, D), jnp.float32),
            pltpu.VMEM((K, D), jnp.float32),
            pltpu.VMEM_SHARED((N_PAD, D), jnp.float32),
        ] + [pltpu.SemaphoreType.DMA] * 16,
    )(_sc_agg_body)


def _sc_agg_body(xt_hbm, packed_hbm, w_hbm, out_hbm,
                 pring, colbuf, rowbuf, wring, b0, b1, b2, b3, acc,
                 g0, g1, g2, g3, s0, s1, s2, s3,
                 w0, w1, w2, w3, p0, p1, p2, p3):
    bufs = (b0, b1, b2, b3)
    gsem = (g0, g1, g2, g3)
    ssem = (s0, s1, s2, s3)
    wsem = (w0, w1, w2, w3)
    psem = (p0, p1, p2, p3)
    c = lax.axis_index("c")
    s = lax.axis_index("s")
    nch = jnp.where(c == 0, NCH0, NCH1)
    base = jnp.where(c == 0, s * NCH0, NS * NCH0 + s * NCH1)

    @pl.loop(0, K)
    def _zrow(j):
        zv = jnp.zeros((16,), jnp.float32)
        for d in range(D // 16):
            b0[j, pl.ds(d * 16, 16)] = zv

    for k in range(ROWS_PER_TILE // K):
        pltpu.sync_copy(b0, acc.at[pl.ds(s * ROWS_PER_TILE + k * K, K)])
    plsc.subcore_barrier()

    def issue_packed(q, r):
        pltpu.async_copy(packed_hbm.at[base + q], pring.at[r], psem[r])
        pltpu.async_copy(w_hbm.at[base + q], wring.at[r], wsem[r])

    def unpack(r):
        for g in range(K // 16):
            sl = pl.ds(g * 16, 16)
            v = pring[r, 0, sl]
            colbuf[r, sl] = lax.bitwise_and(v, 0xFFFF)
            rowbuf[r, sl] = lax.shift_right_logical(v, 16)

    def issue_gather(q, r):
        pltpu.async_copy(xt_hbm.at[colbuf.at[r]], bufs[r], gsem[r])

    def wait_packed(r):
        pltpu.make_async_copy(packed_hbm.at[base], pring.at[r], psem[r]).wait()

    def wait_w(r):
        pltpu.make_async_copy(w_hbm.at[base], wring.at[r], wsem[r]).wait()

    def wait_gather(r):
        pltpu.make_async_copy(xt_hbm.at[colbuf.at[r]], bufs[r], gsem[r]).wait()

    def wait_scatter(r):
        pltpu.make_async_copy(bufs[r], acc.at[rowbuf.at[r]], ssem[r]).wait()

    for p in range(3):
        issue_packed(p, p)
    for p in range(2):
        wait_packed(p)
        unpack(p)
        issue_gather(p, p)

    @pl.loop(0, nch, step=4)
    def _group(i0):
        for b in range(4):
            i = i0 + b
            r2 = (b + 2) % 4
            r3 = (b + 3) % 4

            @pl.when(i + 3 < nch)
            def _stream():
                issue_packed(i + 3, r3)

            @pl.when(i + 2 < nch)
            def _prefetch():
                @pl.when(i >= 2)
                def _pf_drain():
                    wait_scatter(r2)
                wait_packed(r2)
                unpack(r2)
                issue_gather(i + 2, r2)

            wait_gather(b)
            wait_w(b)

            @pl.loop(0, K, unroll=4)
            def _edge(j):
                wv = plsc.load_gather(
                    wring, [jnp.full((16,), b, jnp.int32),
                            jnp.full((16,), 0, jnp.int32),
                            jnp.full((16,), j, jnp.int32)])
                for d in range(D // 16):
                    sl = pl.ds(d * 16, 16)
                    bufs[b][j, sl] = bufs[b][j, sl] * wv

            pltpu.async_copy(bufs[b], acc.at[rowbuf.at[b]], ssem[b], add=True)

    for b in range(4):
        wait_scatter(b)
    plsc.subcore_barrier()
    pltpu.sync_copy(acc.at[pl.ds(s * ROWS_PER_TILE, ROWS_PER_TILE)],
                    out_hbm.at[c, pl.ds(s * ROWS_PER_TILE, ROWS_PER_TILE)])


def kernel(x, edge_index, edge_weight, W, b):
    xt = _stage_a(x, W, b.reshape(1, D))
    pad = E_PAD - E
    row = jnp.concatenate([edge_index[0], jnp.zeros((pad,), jnp.int32)])
    col = jnp.concatenate([edge_index[1], jnp.zeros((pad,), jnp.int32)])
    w = jnp.concatenate([edge_weight, jnp.zeros((pad,), jnp.float32)])
    packed = (col | (row << 16)).reshape(TOTCH, 1, K)
    w3 = w.reshape(TOTCH, 1, K)
    parts = _build_sc_agg()(xt, packed, w3)
    return _stage_c(parts)


# speedup vs baseline: 2.0971x; 1.0219x over previous
import functools

import jax
import jax.numpy as jnp
from jax import lax
from jax.experimental import pallas as pl
from jax.experimental.pallas import tpu as pltpu
from jax.experimental.pallas import tpu_sc as plsc

N = 10000
D = 128
E = 320000
MIN_NORM = 1e-15
PROJ_EPS = 4e-3
MAXNORM = 1.0 - PROJ_EPS

NC = 2
NS = 16
K = 64
NCH0 = 272
NCH1 = 44
TOTCH = NS * (NCH0 + NCH1)
E_PAD = TOTCH * K
N_PAD = 10240
ROWS_PER_TILE = N_PAD // NS


def _artanh(v):
    v = jnp.clip(v, -1.0 + 1e-7, 1.0 - 1e-7)
    return 0.5 * jnp.log((1.0 + v) / (1.0 - v))


def _norm(x):
    return jnp.maximum(jnp.sqrt(jnp.sum(x * x, axis=-1, keepdims=True)), MIN_NORM)


def _proj(x):
    norm = _norm(x)
    return jnp.where(norm > MAXNORM, x / norm * MAXNORM, x)


def _expmap0(u):
    un = _norm(u)
    return jnp.tanh(un) * u / un


def _logmap0(p):
    pn = _norm(p)
    return _artanh(pn) * p / pn


def _stage_a_body(x_ref, w_ref, b_ref, o_ref):
    x = x_ref[...]
    W = w_ref[...]
    b = b_ref[...]
    xn = _norm(x)
    mx = lax.dot_general(x, W, (((1,), (1,)), ((), ())),
                         preferred_element_type=jnp.float32)
    mxn = _norm(mx)
    res = jnp.tanh(mxn / xn * _artanh(xn)) * mx / mxn
    res = jnp.where(jnp.all(mx == 0.0, axis=-1, keepdims=True),
                    jnp.zeros_like(res), res)
    res = _proj(res)
    hyp_bias = _proj(_expmap0(b))
    x2 = jnp.sum(res * res, axis=-1, keepdims=True)
    y2 = jnp.sum(hyp_bias * hyp_bias, axis=-1, keepdims=True)
    xy = jnp.sum(res * hyp_bias, axis=-1, keepdims=True)
    num = (1.0 + 2.0 * xy + y2) * res + (1.0 - x2) * hyp_bias
    den = 1.0 + 2.0 * xy + x2 * y2
    h = _proj(num / jnp.maximum(den, MIN_NORM))
    o_ref[...] = _logmap0(h)


def _stage_c_body(a_ref, o_ref):
    s = a_ref[0] + a_ref[1]
    h = _proj(_expmap0(s))
    xt = jnp.maximum(_logmap0(h), 0.0)
    o_ref[...] = _proj(_expmap0(xt))


_BLK = 2000


def _stage_a(x, W, b2d):
    return pl.pallas_call(
        _stage_a_body,
        grid=(N // _BLK,),
        in_specs=[
            pl.BlockSpec((_BLK, D), lambda i: (i, 0)),
            pl.BlockSpec((D, D), lambda i: (0, 0)),
            pl.BlockSpec((1, D), lambda i: (0, 0)),
        ],
        out_specs=pl.BlockSpec((_BLK, D), lambda i: (i, 0)),
        out_shape=jax.ShapeDtypeStruct((N, D), jnp.float32),
    )(x, W, b2d)


def _stage_c(parts):
    return pl.pallas_call(
        _stage_c_body,
        grid=(N // _BLK,),
        in_specs=[pl.BlockSpec((2, _BLK, D), lambda i: (0, i, 0))],
        out_specs=pl.BlockSpec((_BLK, D), lambda i: (i, 0)),
        out_shape=jax.ShapeDtypeStruct((N, D), jnp.float32),
    )(parts)


@functools.cache
def _build_sc_agg():
    mesh = plsc.VectorSubcoreMesh(
        core_axis_name="c", subcore_axis_name="s",
        num_cores=NC, num_subcores=NS)
    return functools.partial(
        pl.kernel,
        out_type=jax.ShapeDtypeStruct((NC, N_PAD, D), jnp.float32),
        mesh=mesh,
        compiler_params=pltpu.CompilerParams(needs_layout_passes=False),
        scratch_types=[
            pltpu.VMEM((4, 2, K), jnp.int32),
            pltpu.VMEM((4, K), jnp.int32),
            pltpu.VMEM((4, K), jnp.int32),
            pltpu.VMEM((K, D), jnp.float32),
            pltpu.VMEM((K, D), jnp.float32),
            pltpu.VMEM((K, D), jnp.float32),
            pltpu.VMEM((K, D), jnp.float32),
            pltpu.VMEM_SHARED((N_PAD, D), jnp.float32),
        ] + [pltpu.SemaphoreType.DMA] * 12,
    )(_sc_agg_body)


def _sc_agg_body(xt_hbm, packed_hbm, out_hbm,
                 pring, colbuf, rowbuf, b0, b1, b2, b3, acc,
                 g0, g1, g2, g3, s0, s1, s2, s3, p0, p1, p2, p3):
    bufs = (b0, b1, b2, b3)
    gsem = (g0, g1, g2, g3)
    ssem = (s0, s1, s2, s3)
    psem = (p0, p1, p2, p3)
    c = lax.axis_index("c")
    s = lax.axis_index("s")
    nch = jnp.where(c == 0, NCH0, NCH1)
    base = jnp.where(c == 0, s * NCH0, NS * NCH0 + s * NCH1)

    def issue_packed(q, r):
        pltpu.async_copy(packed_hbm.at[base + q], pring.at[r], psem[r])

    def unpack(r):
        for g in range(K // 16):
            sl = pl.ds(g * 16, 16)
            v = pring[r, 0, sl]
            colbuf[r, sl] = lax.bitwise_and(v, 0xFFFF)
            rowbuf[r, sl] = lax.shift_right_logical(v, 16)

    def issue_gather(q, r):
        pltpu.async_copy(xt_hbm.at[colbuf.at[r]], bufs[r], gsem[r])

    def wait_packed(r):
        pltpu.make_async_copy(packed_hbm.at[base], pring.at[r], psem[r]).wait()

    def wait_gather(r):
        pltpu.make_async_copy(xt_hbm.at[colbuf.at[r]], bufs[r], gsem[r]).wait()

    def wait_scatter(r):
        pltpu.make_async_copy(bufs[r], acc.at[rowbuf.at[r]], ssem[r]).wait()

    for p in range(3):
        issue_packed(p, p)

    @pl.loop(0, K)
    def _zrow(j):
        zv = jnp.zeros((16,), jnp.float32)
        for d in range(D // 16):
            b3[j, pl.ds(d * 16, 16)] = zv

    for p in range(2):
        wait_packed(p)
        unpack(p)
        issue_gather(p, p)

    for k in range(ROWS_PER_TILE // K):
        pltpu.sync_copy(b3, acc.at[pl.ds(s * ROWS_PER_TILE + k * K, K)])
    plsc.subcore_barrier()

    @pl.loop(0, nch, step=4)
    def _group(i0):
        for b in range(4):
            i = i0 + b
            r2 = (b + 2) % 4
            r3 = (b + 3) % 4

            @pl.when(i + 3 < nch)
            def _stream():
                issue_packed(i + 3, r3)

            @pl.when(i + 2 < nch)
            def _prefetch():
                @pl.when(i >= 2)
                def _pf_drain():
                    wait_scatter(r2)
                wait_packed(r2)
                unpack(r2)
                issue_gather(i + 2, r2)

            wait_gather(b)

            @pl.loop(0, K, unroll=4)
            def _edge(j):
                wv = plsc.bitcast(plsc.load_gather(
                    pring, [jnp.full((16,), b, jnp.int32),
                            jnp.full((16,), 1, jnp.int32),
                            jnp.full((16,), j, jnp.int32)]), jnp.float32)
                for d in range(D // 16):
                    sl = pl.ds(d * 16, 16)
                    bufs[b][j, sl] = bufs[b][j, sl] * wv

            pltpu.async_copy(bufs[b], acc.at[rowbuf.at[b]], ssem[b], add=True)

    for b in range(4):
        wait_scatter(b)
    plsc.subcore_barrier()
    pltpu.sync_copy(acc.at[pl.ds(s * ROWS_PER_TILE, ROWS_PER_TILE)],
                    out_hbm.at[c, pl.ds(s * ROWS_PER_TILE, ROWS_PER_TILE)])


def kernel(x, edge_index, edge_weight, W, b):
    xt = _stage_a(x, W, b.reshape(1, D))
    pad = E_PAD - E
    row = jnp.concatenate([edge_index[0], jnp.zeros((pad,), jnp.int32)])
    col = jnp.concatenate([edge_index[1], jnp.zeros((pad,), jnp.int32)])
    w = jnp.concatenate([edge_weight, jnp.zeros((pad,), jnp.float32)])
    packed = (col | (row << 16)).reshape(TOTCH, 1, K)
    wbits = lax.bitcast_convert_type(w, jnp.int32).reshape(TOTCH, 1, K)
    pw = jnp.concatenate([packed, wbits], axis=1)
    parts = _build_sc_agg()(xt, pw)
    return _stage_c(parts)


# speedup vs baseline: 2.3198x; 1.1062x over previous
import functools

import jax
import jax.numpy as jnp
from jax import lax
from jax.experimental import pallas as pl
from jax.experimental.pallas import tpu as pltpu
from jax.experimental.pallas import tpu_sc as plsc

N = 10000
D = 128
E = 320000
MIN_NORM = 1e-15
PROJ_EPS = 4e-3
MAXNORM = 1.0 - PROJ_EPS

NC = 2
NS = 16
K = 80
NCH0 = 220
NCH1 = 32
TOTCH = NS * (NCH0 + NCH1)
E_PAD = TOTCH * K
N_PAD = 10240
ROWS_PER_TILE = N_PAD // NS


def _artanh(v):
    v = jnp.clip(v, -1.0 + 1e-7, 1.0 - 1e-7)
    return 0.5 * jnp.log((1.0 + v) / (1.0 - v))


def _norm(x):
    return jnp.maximum(jnp.sqrt(jnp.sum(x * x, axis=-1, keepdims=True)), MIN_NORM)


def _proj(x):
    norm = _norm(x)
    return jnp.where(norm > MAXNORM, x / norm * MAXNORM, x)


def _expmap0(u):
    un = _norm(u)
    return jnp.tanh(un) * u / un


def _logmap0(p):
    pn = _norm(p)
    return _artanh(pn) * p / pn


def _stage_a_body(x_ref, w_ref, b_ref, o_ref):
    x = x_ref[...]
    W = w_ref[...]
    b = b_ref[...]
    xn = _norm(x)
    mx = lax.dot_general(x, W, (((1,), (1,)), ((), ())),
                         preferred_element_type=jnp.float32)
    mxn = _norm(mx)
    res = jnp.tanh(mxn / xn * _artanh(xn)) * mx / mxn
    res = jnp.where(jnp.all(mx == 0.0, axis=-1, keepdims=True),
                    jnp.zeros_like(res), res)
    res = _proj(res)
    hyp_bias = _proj(_expmap0(b))
    x2 = jnp.sum(res * res, axis=-1, keepdims=True)
    y2 = jnp.sum(hyp_bias * hyp_bias, axis=-1, keepdims=True)
    xy = jnp.sum(res * hyp_bias, axis=-1, keepdims=True)
    num = (1.0 + 2.0 * xy + y2) * res + (1.0 - x2) * hyp_bias
    den = 1.0 + 2.0 * xy + x2 * y2
    h = _proj(num / jnp.maximum(den, MIN_NORM))
    o_ref[...] = _logmap0(h)


def _stage_c_body(a_ref, o_ref):
    s = a_ref[0] + a_ref[1]
    h = _proj(_expmap0(s))
    xt = jnp.maximum(_logmap0(h), 0.0)
    o_ref[...] = _proj(_expmap0(xt))


_BLK = 2000


def _stage_a(x, W, b2d):
    return pl.pallas_call(
        _stage_a_body,
        grid=(N // _BLK,),
        in_specs=[
            pl.BlockSpec((_BLK, D), lambda i: (i, 0)),
            pl.BlockSpec((D, D), lambda i: (0, 0)),
            pl.BlockSpec((1, D), lambda i: (0, 0)),
        ],
        out_specs=pl.BlockSpec((_BLK, D), lambda i: (i, 0)),
        out_shape=jax.ShapeDtypeStruct((N, D), jnp.float32),
    )(x, W, b2d)


def _stage_c(parts):
    return pl.pallas_call(
        _stage_c_body,
        grid=(N // _BLK,),
        in_specs=[pl.BlockSpec((2, _BLK, D), lambda i: (0, i, 0))],
        out_specs=pl.BlockSpec((_BLK, D), lambda i: (i, 0)),
        out_shape=jax.ShapeDtypeStruct((N, D), jnp.float32),
    )(parts)


@functools.cache
def _build_sc_agg():
    mesh = plsc.VectorSubcoreMesh(
        core_axis_name="c", subcore_axis_name="s",
        num_cores=NC, num_subcores=NS)
    return functools.partial(
        pl.kernel,
        out_type=jax.ShapeDtypeStruct((NC, N_PAD, D), jnp.float32),
        mesh=mesh,
        compiler_params=pltpu.CompilerParams(needs_layout_passes=False),
        scratch_types=[
            pltpu.VMEM((4, 2, K), jnp.int32),
            pltpu.VMEM((4, K), jnp.int32),
            pltpu.VMEM((4, K), jnp.int32),
            pltpu.VMEM((K, D), jnp.float32),
            pltpu.VMEM((K, D), jnp.float32),
            pltpu.VMEM((K, D), jnp.float32),
            pltpu.VMEM((K, D), jnp.float32),
            pltpu.VMEM_SHARED((N_PAD, D), jnp.float32),
        ] + [pltpu.SemaphoreType.DMA] * 12,
    )(_sc_agg_body)


def _sc_agg_body(xt_hbm, packed_hbm, out_hbm,
                 pring, colbuf, rowbuf, b0, b1, b2, b3, acc,
                 g0, g1, g2, g3, s0, s1, s2, s3, p0, p1, p2, p3):
    bufs = (b0, b1, b2, b3)
    gsem = (g0, g1, g2, g3)
    ssem = (s0, s1, s2, s3)
    psem = (p0, p1, p2, p3)
    c = lax.axis_index("c")
    s = lax.axis_index("s")
    nch = jnp.where(c == 0, NCH0, NCH1)
    base = jnp.where(c == 0, s * NCH0, NS * NCH0 + s * NCH1)

    def issue_packed(q, r):
        pltpu.async_copy(packed_hbm.at[base + q], pring.at[r], psem[r])

    def unpack(r):
        for g in range(K // 16):
            sl = pl.ds(g * 16, 16)
            v = pring[r, 0, sl]
            colbuf[r, sl] = lax.bitwise_and(v, 0xFFFF)
            rowbuf[r, sl] = lax.shift_right_logical(v, 16)

    def issue_gather(q, r):
        pltpu.async_copy(xt_hbm.at[colbuf.at[r]], bufs[r], gsem[r])

    def wait_packed(r):
        pltpu.make_async_copy(packed_hbm.at[base], pring.at[r], psem[r]).wait()

    def wait_gather(r):
        pltpu.make_async_copy(xt_hbm.at[colbuf.at[r]], bufs[r], gsem[r]).wait()

    def wait_scatter(r):
        pltpu.make_async_copy(bufs[r], acc.at[rowbuf.at[r]], ssem[r]).wait()

    for p in range(3):
        issue_packed(p, p)

    @pl.loop(0, K)
    def _zrow(j):
        zv = jnp.zeros((16,), jnp.float32)
        for d in range(D // 16):
            b3[j, pl.ds(d * 16, 16)] = zv

    for p in range(2):
        wait_packed(p)
        unpack(p)
        issue_gather(p, p)

    for k in range(ROWS_PER_TILE // K):
        pltpu.sync_copy(b3, acc.at[pl.ds(s * ROWS_PER_TILE + k * K, K)])
    plsc.subcore_barrier()

    @pl.loop(0, nch, step=4)
    def _group(i0):
        for b in range(4):
            i = i0 + b
            r2 = (b + 2) % 4
            r3 = (b + 3) % 4

            @pl.when(i + 3 < nch)
            def _stream():
                issue_packed(i + 3, r3)

            @pl.when(i + 2 < nch)
            def _prefetch():
                @pl.when(i >= 2)
                def _pf_drain():
                    wait_scatter(r2)
                wait_packed(r2)
                unpack(r2)
                issue_gather(i + 2, r2)

            wait_gather(b)

            @pl.loop(0, K, unroll=4)
            def _edge(j):
                wv = plsc.bitcast(plsc.load_gather(
                    pring, [jnp.full((16,), b, jnp.int32),
                            jnp.full((16,), 1, jnp.int32),
                            jnp.full((16,), j, jnp.int32)]), jnp.float32)
                for d in range(D // 16):
                    sl = pl.ds(d * 16, 16)
                    bufs[b][j, sl] = bufs[b][j, sl] * wv

            pltpu.async_copy(bufs[b], acc.at[rowbuf.at[b]], ssem[b], add=True)

    for b in range(4):
        wait_scatter(b)
    plsc.subcore_barrier()
    pltpu.sync_copy(acc.at[pl.ds(s * ROWS_PER_TILE, ROWS_PER_TILE)],
                    out_hbm.at[c, pl.ds(s * ROWS_PER_TILE, ROWS_PER_TILE)])


def kernel(x, edge_index, edge_weight, W, b):
    xt = _stage_a(x, W, b.reshape(1, D))
    pad = E_PAD - E
    row = jnp.concatenate([edge_index[0], jnp.zeros((pad,), jnp.int32)])
    col = jnp.concatenate([edge_index[1], jnp.zeros((pad,), jnp.int32)])
    w = jnp.concatenate([edge_weight, jnp.zeros((pad,), jnp.float32)])
    packed = (col | (row << 16)).reshape(TOTCH, 1, K)
    wbits = lax.bitcast_convert_type(w, jnp.int32).reshape(TOTCH, 1, K)
    pw = jnp.concatenate([packed, wbits], axis=1)
    parts = _build_sc_agg()(xt, pw)
    return _stage_c(parts)


# speedup vs baseline: 2.3684x; 1.0210x over previous
import functools

import jax
import jax.numpy as jnp
from jax import lax
from jax.experimental import pallas as pl
from jax.experimental.pallas import tpu as pltpu
from jax.experimental.pallas import tpu_sc as plsc

N = 10000
D = 128
E = 320000
MIN_NORM = 1e-15
PROJ_EPS = 4e-3
MAXNORM = 1.0 - PROJ_EPS

NC = 2
NS = 16
K = 80
NCH0 = 212
NCH1 = 40
TOTCH = NS * (NCH0 + NCH1)
E_PAD = TOTCH * K
N_PAD = 10240
ROWS_PER_TILE = N_PAD // NS


def _artanh(v):
    v = jnp.clip(v, -1.0 + 1e-7, 1.0 - 1e-7)
    return 0.5 * jnp.log((1.0 + v) / (1.0 - v))


def _norm(x):
    return jnp.maximum(jnp.sqrt(jnp.sum(x * x, axis=-1, keepdims=True)), MIN_NORM)


def _proj(x):
    norm = _norm(x)
    return jnp.where(norm > MAXNORM, x / norm * MAXNORM, x)


def _expmap0(u):
    un = _norm(u)
    return jnp.tanh(un) * u / un


def _logmap0(p):
    pn = _norm(p)
    return _artanh(pn) * p / pn


def _stage_a_body(x_ref, w_ref, b_ref, o_ref):
    x = x_ref[...]
    W = w_ref[...]
    b = b_ref[...]
    xn = _norm(x)
    mx = lax.dot_general(x, W, (((1,), (1,)), ((), ())),
                         preferred_element_type=jnp.float32)
    mxn = _norm(mx)
    res = jnp.tanh(mxn / xn * _artanh(xn)) * mx / mxn
    res = jnp.where(jnp.all(mx == 0.0, axis=-1, keepdims=True),
                    jnp.zeros_like(res), res)
    res = _proj(res)
    hyp_bias = _proj(_expmap0(b))
    x2 = jnp.sum(res * res, axis=-1, keepdims=True)
    y2 = jnp.sum(hyp_bias * hyp_bias, axis=-1, keepdims=True)
    xy = jnp.sum(res * hyp_bias, axis=-1, keepdims=True)
    num = (1.0 + 2.0 * xy + y2) * res + (1.0 - x2) * hyp_bias
    den = 1.0 + 2.0 * xy + x2 * y2
    h = _proj(num / jnp.maximum(den, MIN_NORM))
    o_ref[...] = _logmap0(h)


def _stage_c_body(a_ref, o_ref):
    s = a_ref[0] + a_ref[1]
    h = _proj(_expmap0(s))
    xt = jnp.maximum(_logmap0(h), 0.0)
    o_ref[...] = _proj(_expmap0(xt))


_BLK = 2000


def _stage_a(x, W, b2d):
    return pl.pallas_call(
        _stage_a_body,
        grid=(N // _BLK,),
        in_specs=[
            pl.BlockSpec((_BLK, D), lambda i: (i, 0)),
            pl.BlockSpec((D, D), lambda i: (0, 0)),
            pl.BlockSpec((1, D), lambda i: (0, 0)),
        ],
        out_specs=pl.BlockSpec((_BLK, D), lambda i: (i, 0)),
        out_shape=jax.ShapeDtypeStruct((N, D), jnp.float32),
    )(x, W, b2d)


def _stage_c(parts):
    return pl.pallas_call(
        _stage_c_body,
        grid=(N // _BLK,),
        in_specs=[pl.BlockSpec((2, _BLK, D), lambda i: (0, i, 0))],
        out_specs=pl.BlockSpec((_BLK, D), lambda i: (i, 0)),
        out_shape=jax.ShapeDtypeStruct((N, D), jnp.float32),
    )(parts)


@functools.cache
def _build_sc_agg():
    mesh = plsc.VectorSubcoreMesh(
        core_axis_name="c", subcore_axis_name="s",
        num_cores=NC, num_subcores=NS)
    return functools.partial(
        pl.kernel,
        out_type=jax.ShapeDtypeStruct((NC, N_PAD, D), jnp.float32),
        mesh=mesh,
        compiler_params=pltpu.CompilerParams(needs_layout_passes=False),
        scratch_types=[
            pltpu.VMEM((4, 2, K), jnp.int32),
            pltpu.VMEM((4, K), jnp.int32),
            pltpu.VMEM((4, K), jnp.int32),
            pltpu.VMEM((K, D), jnp.float32),
            pltpu.VMEM((K, D), jnp.float32),
            pltpu.VMEM((K, D), jnp.float32),
            pltpu.VMEM((K, D), jnp.float32),
            pltpu.VMEM_SHARED((N_PAD, D), jnp.float32),
        ] + [pltpu.SemaphoreType.DMA] * 12,
    )(_sc_agg_body)


def _sc_agg_body(xt_hbm, packed_hbm, out_hbm,
                 pring, colbuf, rowbuf, b0, b1, b2, b3, acc,
                 g0, g1, g2, g3, s0, s1, s2, s3, p0, p1, p2, p3):
    bufs = (b0, b1, b2, b3)
    gsem = (g0, g1, g2, g3)
    ssem = (s0, s1, s2, s3)
    psem = (p0, p1, p2, p3)
    c = lax.axis_index("c")
    s = lax.axis_index("s")
    nch = jnp.where(c == 0, NCH0, NCH1)
    base = jnp.where(c == 0, s * NCH0, NS * NCH0 + s * NCH1)

    def issue_packed(q, r):
        pltpu.async_copy(packed_hbm.at[base + q], pring.at[r], psem[r])

    def unpack(r):
        for g in range(K // 16):
            sl = pl.ds(g * 16, 16)
            v = pring[r, 0, sl]
            colbuf[r, sl] = lax.bitwise_and(v, 0xFFFF)
            rowbuf[r, sl] = lax.shift_right_logical(v, 16)

    def issue_gather(q, r):
        pltpu.async_copy(xt_hbm.at[colbuf.at[r]], bufs[r], gsem[r])

    def wait_packed(r):
        pltpu.make_async_copy(packed_hbm.at[base], pring.at[r], psem[r]).wait()

    def wait_gather(r):
        pltpu.make_async_copy(xt_hbm.at[colbuf.at[r]], bufs[r], gsem[r]).wait()

    def wait_scatter(r):
        pltpu.make_async_copy(bufs[r], acc.at[rowbuf.at[r]], ssem[r]).wait()

    for p in range(3):
        issue_packed(p, p)

    @pl.loop(0, K)
    def _zrow(j):
        zv = jnp.zeros((16,), jnp.float32)
        for d in range(D // 16):
            b3[j, pl.ds(d * 16, 16)] = zv

    for p in range(2):
        wait_packed(p)
        unpack(p)
        issue_gather(p, p)

    for k in range(ROWS_PER_TILE // K):
        pltpu.sync_copy(b3, acc.at[pl.ds(s * ROWS_PER_TILE + k * K, K)])
    _rem = ROWS_PER_TILE % K
    if _rem:
        pltpu.sync_copy(
            b3.at[pl.ds(0, _rem)],
            acc.at[pl.ds(s * ROWS_PER_TILE + (ROWS_PER_TILE // K) * K, _rem)])
    plsc.subcore_barrier()

    @pl.loop(0, nch, step=4)
    def _group(i0):
        for b in range(4):
            i = i0 + b
            r2 = (b + 2) % 4
            r3 = (b + 3) % 4

            @pl.when(i + 3 < nch)
            def _stream():
                issue_packed(i + 3, r3)

            @pl.when(i + 2 < nch)
            def _prefetch():
                @pl.when(i >= 2)
                def _pf_drain():
                    wait_scatter(r2)
                wait_packed(r2)
                unpack(r2)
                issue_gather(i + 2, r2)

            wait_gather(b)

            @pl.loop(0, K, unroll=4)
            def _edge(j):
                wv = plsc.bitcast(plsc.load_gather(
                    pring, [jnp.full((16,), b, jnp.int32),
                            jnp.full((16,), 1, jnp.int32),
                            jnp.full((16,), j, jnp.int32)]), jnp.float32)
                for d in range(D // 16):
                    sl = pl.ds(d * 16, 16)
                    bufs[b][j, sl] = bufs[b][j, sl] * wv

            pltpu.async_copy(bufs[b], acc.at[rowbuf.at[b]], ssem[b], add=True)

    for b in range(4):
        wait_scatter(b)
    plsc.subcore_barrier()
    pltpu.sync_copy(acc.at[pl.ds(s * ROWS_PER_TILE, ROWS_PER_TILE)],
                    out_hbm.at[c, pl.ds(s * ROWS_PER_TILE, ROWS_PER_TILE)])


def kernel(x, edge_index, edge_weight, W, b):
    xt = _stage_a(x, W, b.reshape(1, D))
    pad = E_PAD - E
    row = jnp.concatenate([edge_index[0], jnp.zeros((pad,), jnp.int32)])
    col = jnp.concatenate([edge_index[1], jnp.zeros((pad,), jnp.int32)])
    w = jnp.concatenate([edge_weight, jnp.zeros((pad,), jnp.float32)])
    packed = (col | (row << 16)).reshape(TOTCH, 1, K)
    wbits = lax.bitcast_convert_type(w, jnp.int32).reshape(TOTCH, 1, K)
    pw = jnp.concatenate([packed, wbits], axis=1)
    parts = _build_sc_agg()(xt, pw)
    return _stage_c(parts)
